# Initial kernel scaffold; baseline (speedup 1.0000x reference)
import functools

import jax
import jax.numpy as jnp
from jax import lax
from jax.experimental import pallas as pl
from jax.experimental.pallas import tpu as pltpu
from jax.experimental.pallas import tpu_sc as plsc

_EPSV = 1e-5
_G = 256
_C = 80
_NC = 2
_NS = 16
_RB = 1000


def _make_deg(N, E):
  nch = E // (_NS * _C)
  rpt = N // _NS
  mesh = plsc.VectorSubcoreMesh(core_axis_name="c", subcore_axis_name="s")

  @functools.partial(
      pl.kernel, mesh=mesh,
      out_type=jax.ShapeDtypeStruct((N, 16), jnp.float32),
      scratch_types=[
          pltpu.VMEM((nch, _C), jnp.int32),
          pltpu.VMEM((_C, 16), jnp.float32),
          pltpu.VMEM_SHARED((N, 16), jnp.float32),
      ],
  )
  def k(dst_hbm, ones_hbm, out_hbm, dstv, ones_v, acc):
    cid = lax.axis_index("c")
    sid = lax.axis_index("s")

    @pl.when(cid == 0)
    def _():
      pltpu.sync_copy(ones_hbm.at[pl.ds(0, _C)], ones_v)
      pltpu.sync_copy(ones_hbm.at[pl.ds(sid * rpt, rpt)],
                      acc.at[pl.ds(sid * rpt, rpt)])
      pltpu.sync_copy(dst_hbm.at[sid], dstv)
      plsc.subcore_barrier()

      def body(ci, carry):
        pltpu.sync_copy(ones_v, acc.at[dstv.at[ci]], add=True)
        return carry

      lax.fori_loop(0, nch, body, 0, unroll=False)
      plsc.subcore_barrier()
      pltpu.sync_copy(acc.at[pl.ds(sid * rpt, rpt)],
                      out_hbm.at[pl.ds(sid * rpt, rpt)])

  return k


def _make_scatter(N, E, D):
  nw = _NC * _NS
  epw = E // nw
  nch = epw // _C
  rpt = N // _NS
  mesh = plsc.VectorSubcoreMesh(core_axis_name="c", subcore_axis_name="s")

  @functools.partial(
      pl.kernel, mesh=mesh,
      out_type=jax.ShapeDtypeStruct((_NC, N, D), jnp.float32),
      scratch_types=[
          pltpu.VMEM((nch, _C), jnp.int32),
          pltpu.VMEM((nch, _C), jnp.int32),
          pltpu.VMEM((_C, D), jnp.float32),
          pltpu.VMEM_SHARED((N, D), jnp.float32),
          pltpu.SemaphoreType.DMA,
      ],
  )
  def k(src_hbm, dst_hbm, hs_hbm, zero_hbm, out_hbm, srcv, dstv, rows, acc,
        sem):
    cid = lax.axis_index("c")
    sid = lax.axis_index("s")
    wid = cid * _NS + sid

    @pl.when(cid == 0)
    def _():
      pltpu.sync_copy(hs_hbm.at[pl.ds(sid * rpt, rpt)],
                      acc.at[pl.ds(sid * rpt, rpt)])

    @pl.when(cid == 1)
    def _():
      pltpu.sync_copy(zero_hbm.at[pl.ds(sid * rpt, rpt)],
                      acc.at[pl.ds(sid * rpt, rpt)])

    pltpu.sync_copy(src_hbm.at[wid], srcv)
    pltpu.sync_copy(dst_hbm.at[wid], dstv)
    plsc.subcore_barrier()

    def body(ci, carry):
      pltpu.async_copy(hs_hbm.at[srcv.at[ci]], rows, sem).wait()
      pltpu.sync_copy(rows, acc.at[dstv.at[ci]], add=True)
      return carry

    lax.fori_loop(0, nch, body, 0, unroll=False)
    plsc.subcore_barrier()
    pltpu.sync_copy(acc.at[pl.ds(sid * rpt, rpt)],
                    out_hbm.at[cid, pl.ds(sid * rpt, rpt)])

  return k


def _mm_body(x_ref, w_ref, deg_ref, o_ref):
  dinv = lax.rsqrt(deg_ref[...][:, 0:1])
  h = jnp.dot(x_ref[...], w_ref[...], preferred_element_type=jnp.float32)
  o_ref[...] = h * dinv


def _mm_bn_body(x_ref, w_ref, deg_ref, cs_ref, css_ref, g_ref, be_ref, o_ref,
                *, inv_n):
  m = cs_ref[...] * inv_n
  v = css_ref[...] * inv_n - m * m
  s = g_ref[...] * lax.rsqrt(v + _EPSV)
  t = be_ref[...] - m * s
  xb = x_ref[...] * s + t
  dinv = lax.rsqrt(deg_ref[...][:, 0:1])
  h = jnp.dot(xb, w_ref[...], preferred_element_type=jnp.float32)
  o_ref[...] = h * dinv


def _post_body(p_ref, deg_ref, b_ref, r_ref, cs_ref, css_ref):
  agg = p_ref[0] + p_ref[1]
  dinv = lax.rsqrt(deg_ref[...][:, 0:1])
  rr = jnp.maximum(agg * dinv + b_ref[...], 0.0)
  r_ref[...] = rr

  @pl.when(pl.program_id(0) == 0)
  def _():
    cs_ref[...] = jnp.zeros_like(cs_ref)
    css_ref[...] = jnp.zeros_like(css_ref)

  cs_ref[...] += jnp.sum(rr, axis=0, keepdims=True)
  css_ref[...] += jnp.sum(rr * rr, axis=0, keepdims=True)


def _post4_body(p_ref, deg_ref, b_ref, batch_ref, pooled_ref, cnt_ref, cs_ref,
                css_ref):
  agg = p_ref[0] + p_ref[1]
  dinv = lax.rsqrt(deg_ref[...][:, 0:1])
  rr = jnp.maximum(agg * dinv + b_ref[...], 0.0)
  bt = batch_ref[0]
  gi = lax.broadcasted_iota(jnp.int32, (_G, bt.shape[1]), 0)
  mt = (gi == bt).astype(jnp.float32)
  pp = jnp.dot(mt, rr, preferred_element_type=jnp.float32)
  cp = jnp.sum(mt, axis=1, keepdims=True)

  @pl.when(pl.program_id(0) == 0)
  def _():
    pooled_ref[...] = jnp.zeros_like(pooled_ref)
    cnt_ref[...] = jnp.zeros_like(cnt_ref)
    cs_ref[...] = jnp.zeros_like(cs_ref)
    css_ref[...] = jnp.zeros_like(css_ref)

  pooled_ref[...] += pp
  cnt_ref[...] += cp
  cs_ref[...] += jnp.sum(rr, axis=0, keepdims=True)
  css_ref[...] += jnp.sum(rr * rr, axis=0, keepdims=True)


def _head_body(pooled_ref, cnt_ref, cs_ref, css_ref, g4_ref, be4_ref,
               fw1_ref, fb1_ref, fg1_ref, fbe1_ref, fw2_ref, fb2_ref,
               fw3_ref, fb3_ref, o_ref, *, inv_n, g):
  m = cs_ref[...] * inv_n
  v = css_ref[...] * inv_n - m * m
  s = g4_ref[...] * lax.rsqrt(v + _EPSV)
  t = be4_ref[...] - m * s
  pooled = pooled_ref[...] * s + cnt_ref[...] * t
  z = jnp.dot(pooled, fw1_ref[...], preferred_element_type=jnp.float32)
  z = jnp.maximum(z + fb1_ref[...], 0.0)
  m1 = jnp.sum(z, axis=0, keepdims=True) * (1.0 / g)
  v1 = jnp.sum(z * z, axis=0, keepdims=True) * (1.0 / g) - m1 * m1
  z = (z - m1) * lax.rsqrt(v1 + _EPSV) * fg1_ref[...] + fbe1_ref[...]
  z = jnp.dot(z, fw2_ref[...], preferred_element_type=jnp.float32)
  z = jnp.maximum(z + fb2_ref[...], 0.0)
  o_ref[...] = (jnp.dot(z, fw3_ref[...], preferred_element_type=jnp.float32)
                + fb3_ref[...])


def _row_spec(rb, d):
  return pl.BlockSpec((rb, d), lambda i: (i, 0))


def _full_spec(shape):
  nd = len(shape)
  return pl.BlockSpec(shape, lambda i: (0,) * nd)


def _mm(x, w, deg):
  n, d = x.shape
  return pl.pallas_call(
      _mm_body,
      grid=(n // _RB,),
      in_specs=[_row_spec(_RB, d), _full_spec(w.shape), _row_spec(_RB, 16)],
      out_specs=_row_spec(_RB, w.shape[1]),
      out_shape=jax.ShapeDtypeStruct((n, w.shape[1]), jnp.float32),
  )(x, w, deg)


def _mm_bn(x, w, deg, cs, css, gr, ber):
  n, d = x.shape
  return pl.pallas_call(
      functools.partial(_mm_bn_body, inv_n=1.0 / n),
      grid=(n // _RB,),
      in_specs=[_row_spec(_RB, d), _full_spec(w.shape), _row_spec(_RB, 16),
                _full_spec(cs.shape), _full_spec(css.shape),
                _full_spec(gr.shape), _full_spec(ber.shape)],
      out_specs=_row_spec(_RB, w.shape[1]),
      out_shape=jax.ShapeDtypeStruct((n, w.shape[1]), jnp.float32),
  )(x, w, deg, cs, css, gr, ber)


def _post(p, deg, br):
  _, n, d = p.shape
  return pl.pallas_call(
      _post_body,
      grid=(n // _RB,),
      in_specs=[pl.BlockSpec((2, _RB, d), lambda i: (0, i, 0)),
                _row_spec(_RB, 16), _full_spec(br.shape)],
      out_specs=[_row_spec(_RB, d), _full_spec((1, d)), _full_spec((1, d))],
      out_shape=[jax.ShapeDtypeStruct((n, d), jnp.float32),
                 jax.ShapeDtypeStruct((1, d), jnp.float32),
                 jax.ShapeDtypeStruct((1, d), jnp.float32)],
  )(p, deg, br)


def _post4(p, deg, br, batch3):
  _, n, d = p.shape
  return pl.pallas_call(
      _post4_body,
      grid=(n // _RB,),
      in_specs=[pl.BlockSpec((2, _RB, d), lambda i: (0, i, 0)),
                _row_spec(_RB, 16), _full_spec(br.shape),
                pl.BlockSpec((1, 1, _RB), lambda i: (i, 0, 0))],
      out_specs=[_full_spec((_G, d)), _full_spec((_G, 1)),
                 _full_spec((1, d)), _full_spec((1, d))],
      out_shape=[jax.ShapeDtypeStruct((_G, d), jnp.float32),
                 jax.ShapeDtypeStruct((_G, 1), jnp.float32),
                 jax.ShapeDtypeStruct((1, d), jnp.float32),
                 jax.ShapeDtypeStruct((1, d), jnp.float32)],
  )(p, deg, br, batch3)


def _head(pooled, cnt, cs, css, g4r, be4r, fw1, fb1r, fg1r, fbe1r, fw2, fb2r,
          fw3, fb3r, n_nodes):
  args = [pooled, cnt, cs, css, g4r, be4r, fw1, fb1r, fg1r, fbe1r, fw2, fb2r,
          fw3, fb3r]
  return pl.pallas_call(
      functools.partial(_head_body, inv_n=1.0 / n_nodes, g=float(_G)),
      in_specs=[_full_spec(a.shape) for a in args],
      out_specs=_full_spec((_G, 1)),
      out_shape=jax.ShapeDtypeStruct((_G, 1), jnp.float32),
  )(*args)


def kernel(x, edge_index, batch, W1, b1, g1, be1, W2, b2, g2, be2, W3, b3,
           g3, be3, W4, b4, g4, be4, fW1, fb1, fg1, fbe1, fW2, fb2, fW3, fb3):
  n, d = x.shape
  e = edge_index.shape[1]
  nw = _NC * _NS

  src = edge_index[0].reshape(nw, e // (nw * _C), _C)
  dst = edge_index[1].reshape(nw, e // (nw * _C), _C)
  dst_deg = edge_index[1].reshape(_NS, e // (_NS * _C), _C)
  ones16 = jnp.ones((n, 16), jnp.float32)
  zeros_nd = jnp.zeros((n, d), jnp.float32)
  batch3 = batch.reshape(n // _RB, 1, _RB)
  row = lambda a: a.reshape(1, -1)

  deg = _make_deg(n, e)(dst_deg, ones16)
  scat = _make_scatter(n, e, d)

  hs = _mm(x, W1, deg)
  p = scat(src, dst, hs, zeros_nd)
  r, cs, css = _post(p, deg, row(b1))

  hs = _mm_bn(r, W2, deg, cs, css, row(g1), row(be1))
  p = scat(src, dst, hs, zeros_nd)
  r, cs, css = _post(p, deg, row(b2))

  hs = _mm_bn(r, W3, deg, cs, css, row(g2), row(be2))
  p = scat(src, dst, hs, zeros_nd)
  r, cs, css = _post(p, deg, row(b3))

  hs = _mm_bn(r, W4, deg, cs, css, row(g3), row(be3))
  p = scat(src, dst, hs, zeros_nd)
  pooled, cnt, cs, css = _post4(p, deg, row(b4), batch3)

  return _head(pooled, cnt, cs, css, row(g4), row(be4), fW1, row(fb1),
               row(fg1), row(fbe1), fW2, row(fb2), fW3, row(fb3), n)


# speedup vs baseline: 20.4233x; 20.4233x over previous
import functools

import jax
import jax.numpy as jnp
from jax import lax
from jax.experimental import pallas as pl
from jax.experimental.pallas import tpu as pltpu
from jax.experimental.pallas import tpu_sc as plsc

_EPSV = 1e-5
_G = 256
_C = 80
_NC = 2
_NS = 16
_RB = 1000


def _make_scatter(N, E, D, const_rows=False):
  nw = _NC * _NS
  epw = E // nw
  nch = epw // _C
  rpt = 1000
  nio = N // rpt
  mesh = plsc.VectorSubcoreMesh(core_axis_name="c", subcore_axis_name="s")

  @functools.partial(
      pl.kernel, mesh=mesh,
      out_type=jax.ShapeDtypeStruct((_NC, N, D), jnp.float32),
      scratch_types=[
          pltpu.VMEM((2, _C), jnp.int32),
          pltpu.VMEM((nch, _C), jnp.int32),
          pltpu.VMEM((2, _C, D), jnp.float32),
          pltpu.VMEM_SHARED((N, D), jnp.float32),
          pltpu.SemaphoreType.DMA,
          pltpu.SemaphoreType.DMA,
          pltpu.SemaphoreType.DMA,
          pltpu.SemaphoreType.DMA,
      ],
  )
  def k(src_hbm, dst_hbm, hs_hbm, zero_hbm, out_hbm, srcb, dstv, rows, acc,
        g0, g1, s0, s1):
    cid = lax.axis_index("c")
    sid = lax.axis_index("s")
    wid = cid * _NS + sid
    gsem = (g0, g1)
    ssem = (s0, s1)

    @pl.when((cid == 0) & (sid < nio))
    def _():
      pltpu.sync_copy(hs_hbm.at[pl.ds(sid * rpt, rpt)],
                      acc.at[pl.ds(sid * rpt, rpt)])

    @pl.when((cid == 1) & (sid < nio))
    def _():
      pltpu.sync_copy(zero_hbm.at[pl.ds(sid * rpt, rpt)],
                      acc.at[pl.ds(sid * rpt, rpt)])

    pltpu.sync_copy(dst_hbm.at[wid], dstv)
    plsc.subcore_barrier()

    if const_rows:
      pltpu.sync_copy(hs_hbm.at[pl.ds(0, _C)], rows.at[0])

      def body(ci, carry):
        pltpu.sync_copy(rows.at[0], acc.at[dstv.at[ci]], add=True)
        return carry

      lax.fori_loop(0, nch, body, 0, unroll=False)
    else:
      pltpu.sync_copy(src_hbm.at[wid, 0], srcb.at[0])
      pltpu.sync_copy(src_hbm.at[wid, 1], srcb.at[1])
      pltpu.async_copy(hs_hbm.at[srcb.at[0]], rows.at[0], g0)
      pltpu.async_copy(hs_hbm.at[srcb.at[1]], rows.at[1], g1)

      def body(cp, carry):
        for b in range(2):
          i = cp * 2 + b

          @pl.when(i < nch)
          def _():
            pltpu.make_async_copy(hs_hbm.at[srcb.at[b]], rows.at[b],
                                  gsem[b]).wait()

            @pl.when(i + 2 < nch)
            def _():
              pltpu.async_copy(src_hbm.at[wid, i + 2], srcb.at[b], ssem[b])

            pltpu.sync_copy(rows.at[b], acc.at[dstv.at[i]], add=True)

            @pl.when(i + 2 < nch)
            def _():
              pltpu.make_async_copy(src_hbm.at[wid, i + 2], srcb.at[b],
                                    ssem[b]).wait()
              pltpu.async_copy(hs_hbm.at[srcb.at[b]], rows.at[b], gsem[b])
        return carry

      lax.fori_loop(0, (nch + 1) // 2, body, 0, unroll=False)

    plsc.subcore_barrier()

    @pl.when(sid < nio)
    def _():
      pltpu.sync_copy(acc.at[pl.ds(sid * rpt, rpt)],
                      out_hbm.at[cid, pl.ds(sid * rpt, rpt)])

  return k


def _deg16_body(p_ref, o_ref):
  o_ref[...] = p_ref[0][:, 0:16] + p_ref[1][:, 0:16]


def _deg16(p):
  _, n, d = p.shape
  return pl.pallas_call(
      _deg16_body,
      grid=(n // _RB,),
      in_specs=[pl.BlockSpec((2, _RB, d), lambda i: (0, i, 0))],
      out_specs=_row_spec(_RB, 16),
      out_shape=jax.ShapeDtypeStruct((n, 16), jnp.float32),
  )(p)


def _mm_body(x_ref, w_ref, deg_ref, o_ref):
  dinv = lax.rsqrt(deg_ref[...][:, 0:1])
  h = jnp.dot(x_ref[...], w_ref[...], preferred_element_type=jnp.float32)
  o_ref[...] = h * dinv


def _mm_bn_body(x_ref, w_ref, deg_ref, cs_ref, css_ref, g_ref, be_ref, o_ref,
                *, inv_n):
  m = cs_ref[...] * inv_n
  v = css_ref[...] * inv_n - m * m
  s = g_ref[...] * lax.rsqrt(v + _EPSV)
  t = be_ref[...] - m * s
  xb = x_ref[...] * s + t
  dinv = lax.rsqrt(deg_ref[...][:, 0:1])
  h = jnp.dot(xb, w_ref[...], preferred_element_type=jnp.float32)
  o_ref[...] = h * dinv


def _post_body(p_ref, deg_ref, b_ref, r_ref, cs_ref, css_ref):
  agg = p_ref[0] + p_ref[1]
  dinv = lax.rsqrt(deg_ref[...][:, 0:1])
  rr = jnp.maximum(agg * dinv + b_ref[...], 0.0)
  r_ref[...] = rr

  @pl.when(pl.program_id(0) == 0)
  def _():
    cs_ref[...] = jnp.zeros_like(cs_ref)
    css_ref[...] = jnp.zeros_like(css_ref)

  cs_ref[...] += jnp.sum(rr, axis=0, keepdims=True)
  css_ref[...] += jnp.sum(rr * rr, axis=0, keepdims=True)


def _post4_body(p_ref, deg_ref, b_ref, batch_ref, pooled_ref, cnt_ref, cs_ref,
                css_ref):
  agg = p_ref[0] + p_ref[1]
  dinv = lax.rsqrt(deg_ref[...][:, 0:1])
  rr = jnp.maximum(agg * dinv + b_ref[...], 0.0)
  bt = batch_ref[0]
  gi = lax.broadcasted_iota(jnp.int32, (_G, bt.shape[1]), 0)
  mt = (gi == bt).astype(jnp.float32)
  pp = jnp.dot(mt, rr, preferred_element_type=jnp.float32)
  cp = jnp.sum(mt, axis=1, keepdims=True)

  @pl.when(pl.program_id(0) == 0)
  def _():
    pooled_ref[...] = jnp.zeros_like(pooled_ref)
    cnt_ref[...] = jnp.zeros_like(cnt_ref)
    cs_ref[...] = jnp.zeros_like(cs_ref)
    css_ref[...] = jnp.zeros_like(css_ref)

  pooled_ref[...] += pp
  cnt_ref[...] += cp
  cs_ref[...] += jnp.sum(rr, axis=0, keepdims=True)
  css_ref[...] += jnp.sum(rr * rr, axis=0, keepdims=True)


def _head_body(pooled_ref, cnt_ref, cs_ref, css_ref, g4_ref, be4_ref,
               fw1_ref, fb1_ref, fg1_ref, fbe1_ref, fw2_ref, fb2_ref,
               fw3_ref, fb3_ref, o_ref, *, inv_n, g):
  m = cs_ref[...] * inv_n
  v = css_ref[...] * inv_n - m * m
  s = g4_ref[...] * lax.rsqrt(v + _EPSV)
  t = be4_ref[...] - m * s
  pooled = pooled_ref[...] * s + cnt_ref[...] * t
  z = jnp.dot(pooled, fw1_ref[...], preferred_element_type=jnp.float32)
  z = jnp.maximum(z + fb1_ref[...], 0.0)
  m1 = jnp.sum(z, axis=0, keepdims=True) * (1.0 / g)
  v1 = jnp.sum(z * z, axis=0, keepdims=True) * (1.0 / g) - m1 * m1
  z = (z - m1) * lax.rsqrt(v1 + _EPSV) * fg1_ref[...] + fbe1_ref[...]
  z = jnp.dot(z, fw2_ref[...], preferred_element_type=jnp.float32)
  z = jnp.maximum(z + fb2_ref[...], 0.0)
  o_ref[...] = (jnp.dot(z, fw3_ref[...], preferred_element_type=jnp.float32)
                + fb3_ref[...])


def _row_spec(rb, d):
  return pl.BlockSpec((rb, d), lambda i: (i, 0))


def _full_spec(shape):
  nd = len(shape)
  return pl.BlockSpec(shape, lambda *i: (0,) * nd)


def _mm(x, w, deg):
  n, d = x.shape
  return pl.pallas_call(
      _mm_body,
      grid=(n // _RB,),
      in_specs=[_row_spec(_RB, d), _full_spec(w.shape), _row_spec(_RB, 16)],
      out_specs=_row_spec(_RB, w.shape[1]),
      out_shape=jax.ShapeDtypeStruct((n, w.shape[1]), jnp.float32),
  )(x, w, deg)


def _mm_bn(x, w, deg, cs, css, gr, ber):
  n, d = x.shape
  return pl.pallas_call(
      functools.partial(_mm_bn_body, inv_n=1.0 / n),
      grid=(n // _RB,),
      in_specs=[_row_spec(_RB, d), _full_spec(w.shape), _row_spec(_RB, 16),
                _full_spec(cs.shape), _full_spec(css.shape),
                _full_spec(gr.shape), _full_spec(ber.shape)],
      out_specs=_row_spec(_RB, w.shape[1]),
      out_shape=jax.ShapeDtypeStruct((n, w.shape[1]), jnp.float32),
  )(x, w, deg, cs, css, gr, ber)


def _post(p, deg, br):
  _, n, d = p.shape
  return pl.pallas_call(
      _post_body,
      grid=(n // _RB,),
      in_specs=[pl.BlockSpec((2, _RB, d), lambda i: (0, i, 0)),
                _row_spec(_RB, 16), _full_spec(br.shape)],
      out_specs=[_row_spec(_RB, d), _full_spec((1, d)), _full_spec((1, d))],
      out_shape=[jax.ShapeDtypeStruct((n, d), jnp.float32),
                 jax.ShapeDtypeStruct((1, d), jnp.float32),
                 jax.ShapeDtypeStruct((1, d), jnp.float32)],
  )(p, deg, br)


def _post4(p, deg, br, batch3):
  _, n, d = p.shape
  return pl.pallas_call(
      _post4_body,
      grid=(n // _RB,),
      in_specs=[pl.BlockSpec((2, _RB, d), lambda i: (0, i, 0)),
                _row_spec(_RB, 16), _full_spec(br.shape),
                pl.BlockSpec((1, 1, _RB), lambda i: (i, 0, 0))],
      out_specs=[_full_spec((_G, d)), _full_spec((_G, 1)),
                 _full_spec((1, d)), _full_spec((1, d))],
      out_shape=[jax.ShapeDtypeStruct((_G, d), jnp.float32),
                 jax.ShapeDtypeStruct((_G, 1), jnp.float32),
                 jax.ShapeDtypeStruct((1, d), jnp.float32),
                 jax.ShapeDtypeStruct((1, d), jnp.float32)],
  )(p, deg, br, batch3)


def _head(pooled, cnt, cs, css, g4r, be4r, fw1, fb1r, fg1r, fbe1r, fw2, fb2r,
          fw3, fb3r, n_nodes):
  args = [pooled, cnt, cs, css, g4r, be4r, fw1, fb1r, fg1r, fbe1r, fw2, fb2r,
          fw3, fb3r]
  return pl.pallas_call(
      functools.partial(_head_body, inv_n=1.0 / n_nodes, g=float(_G)),
      in_specs=[_full_spec(a.shape) for a in args],
      out_specs=_full_spec((_G, 1)),
      out_shape=jax.ShapeDtypeStruct((_G, 1), jnp.float32),
  )(*args)


def kernel(x, edge_index, batch, W1, b1, g1, be1, W2, b2, g2, be2, W3, b3,
           g3, be3, W4, b4, g4, be4, fW1, fb1, fg1, fbe1, fW2, fb2, fW3, fb3):
  n, d = x.shape
  e = edge_index.shape[1]
  nw = _NC * _NS

  src = edge_index[0].reshape(nw, e // (nw * _C), _C)
  dst = edge_index[1].reshape(nw, e // (nw * _C), _C)
  ones_nd = jnp.ones((n, d), jnp.float32)
  zeros_nd = jnp.zeros((n, d), jnp.float32)
  batch3 = batch.reshape(n // _RB, 1, _RB)
  row = lambda a: a.reshape(1, -1)

  scat = _make_scatter(n, e, d)
  deg = _deg16(_make_scatter(n, e, d, const_rows=True)(
      src, dst, ones_nd, zeros_nd))

  hs = _mm(x, W1, deg)
  p = scat(src, dst, hs, zeros_nd)
  r, cs, css = _post(p, deg, row(b1))

  hs = _mm_bn(r, W2, deg, cs, css, row(g1), row(be1))
  p = scat(src, dst, hs, zeros_nd)
  r, cs, css = _post(p, deg, row(b2))

  hs = _mm_bn(r, W3, deg, cs, css, row(g2), row(be2))
  p = scat(src, dst, hs, zeros_nd)
  r, cs, css = _post(p, deg, row(b3))

  hs = _mm_bn(r, W4, deg, cs, css, row(g3), row(be3))
  p = scat(src, dst, hs, zeros_nd)
  pooled, cnt, cs, css = _post4(p, deg, row(b4), batch3)

  return _head(pooled, cnt, cs, css, row(g4), row(be4), fW1, row(fb1),
               row(fg1), row(fbe1), fW2, row(fb2), fW3, row(fb3), n)


# speedup vs baseline: 24.5971x; 1.2044x over previous
import functools

import jax
import jax.numpy as jnp
from jax import lax
from jax.experimental import pallas as pl
from jax.experimental.pallas import tpu as pltpu
from jax.experimental.pallas import tpu_sc as plsc

_EPSV = 1e-5
_G = 256
_C = 125
_NC = 2
_NS = 16
_RB = 1000


def _make_deg(N, E):
  nw = _NC * _NS
  nch = E // (nw * _C)
  rpt = 1000
  nio = N // rpt
  mesh = plsc.VectorSubcoreMesh(core_axis_name="c", subcore_axis_name="s")

  @functools.partial(
      pl.kernel, mesh=mesh,
      out_type=jax.ShapeDtypeStruct((_NC, N, 8), jnp.float32),
      scratch_types=[
          pltpu.VMEM((nch, _C), jnp.int32),
          pltpu.VMEM((_C, 8), jnp.float32),
          pltpu.VMEM_SHARED((N, 8), jnp.float32),
      ],
      compiler_params=pltpu.CompilerParams(use_tc_tiling_on_sc=False),
  )
  def k(dst_hbm, ones_hbm, zero_hbm, out_hbm, dstv, ones_v, acc):
    cid = lax.axis_index("c")
    sid = lax.axis_index("s")
    wid = cid * _NS + sid

    @pl.when((cid == 0) & (sid < nio))
    def _():
      pltpu.sync_copy(ones_hbm.at[pl.ds(sid * rpt, rpt)],
                      acc.at[pl.ds(sid * rpt, rpt)])

    @pl.when((cid == 1) & (sid < nio))
    def _():
      pltpu.sync_copy(zero_hbm.at[pl.ds(sid * rpt, rpt)],
                      acc.at[pl.ds(sid * rpt, rpt)])

    pltpu.sync_copy(dst_hbm.at[wid], dstv)
    pltpu.sync_copy(ones_hbm.at[pl.ds(0, _C)], ones_v)
    plsc.subcore_barrier()

    def body(ci, carry):
      pltpu.sync_copy(ones_v, acc.at[dstv.at[ci]], add=True)
      return carry

    lax.fori_loop(0, nch, body, 0, unroll=False)
    plsc.subcore_barrier()

    @pl.when(sid < nio)
    def _():
      pltpu.sync_copy(acc.at[pl.ds(sid * rpt, rpt)],
                      out_hbm.at[cid, pl.ds(sid * rpt, rpt)])

  return k


def _make_scatter(N, E, D):
  nw = _NC * _NS
  epw = E // nw
  nch = epw // _C
  rpt = 1000
  nio = N // rpt
  mesh = plsc.VectorSubcoreMesh(core_axis_name="c", subcore_axis_name="s")

  @functools.partial(
      pl.kernel, mesh=mesh,
      out_type=jax.ShapeDtypeStruct((_NC, N, D), jnp.float32),
      scratch_types=[
          pltpu.VMEM((2, _C), jnp.int32),
          pltpu.VMEM((nch, _C), jnp.int32),
          pltpu.VMEM((2, _C, D), jnp.float32),
          pltpu.VMEM_SHARED((N, D), jnp.float32),
          pltpu.SemaphoreType.DMA,
          pltpu.SemaphoreType.DMA,
          pltpu.SemaphoreType.DMA,
          pltpu.SemaphoreType.DMA,
      ],
  )
  def k(src_hbm, dst_hbm, hs_hbm, zero_hbm, out_hbm, srcb, dstv, rows, acc,
        g0, g1, s0, s1):
    cid = lax.axis_index("c")
    sid = lax.axis_index("s")
    wid = cid * _NS + sid
    gsem = (g0, g1)
    ssem = (s0, s1)

    @pl.when((cid == 0) & (sid < nio))
    def _():
      pltpu.sync_copy(hs_hbm.at[pl.ds(sid * rpt, rpt)],
                      acc.at[pl.ds(sid * rpt, rpt)])

    @pl.when((cid == 1) & (sid < nio))
    def _():
      pltpu.sync_copy(zero_hbm.at[pl.ds(sid * rpt, rpt)],
                      acc.at[pl.ds(sid * rpt, rpt)])

    pltpu.sync_copy(dst_hbm.at[wid], dstv)
    plsc.subcore_barrier()

    pltpu.sync_copy(src_hbm.at[wid, 0], srcb.at[0])
    pltpu.sync_copy(src_hbm.at[wid, 1], srcb.at[1])
    pltpu.async_copy(hs_hbm.at[srcb.at[0]], rows.at[0], g0)
    pltpu.async_copy(hs_hbm.at[srcb.at[1]], rows.at[1], g1)

    def body(cp, carry):
      for b in range(2):
        i = cp * 2 + b

        @pl.when(i < nch)
        def _():
          pltpu.make_async_copy(hs_hbm.at[srcb.at[b]], rows.at[b],
                                gsem[b]).wait()

          @pl.when(i + 2 < nch)
          def _():
            pltpu.async_copy(src_hbm.at[wid, i + 2], srcb.at[b], ssem[b])

          pltpu.sync_copy(rows.at[b], acc.at[dstv.at[i]], add=True)

          @pl.when(i + 2 < nch)
          def _():
            pltpu.make_async_copy(src_hbm.at[wid, i + 2], srcb.at[b],
                                  ssem[b]).wait()
            pltpu.async_copy(hs_hbm.at[srcb.at[b]], rows.at[b], gsem[b])
      return carry

    lax.fori_loop(0, (nch + 1) // 2, body, 0, unroll=False)

    plsc.subcore_barrier()

    @pl.when(sid < nio)
    def _():
      pltpu.sync_copy(acc.at[pl.ds(sid * rpt, rpt)],
                      out_hbm.at[cid, pl.ds(sid * rpt, rpt)])

  return k


def _deg8sum_body(p_ref, o_ref):
  o_ref[...] = p_ref[0] + p_ref[1]


def _deg8sum(p):
  _, n, d = p.shape
  return pl.pallas_call(
      _deg8sum_body,
      grid=(n // _RB,),
      in_specs=[pl.BlockSpec((2, _RB, d), lambda i: (0, i, 0))],
      out_specs=_row_spec(_RB, d),
      out_shape=jax.ShapeDtypeStruct((n, d), jnp.float32),
  )(p)


def _mm_body(x_ref, w_ref, deg_ref, o_ref):
  dinv = lax.rsqrt(deg_ref[...][:, 0:1])
  h = jnp.dot(x_ref[...], w_ref[...], preferred_element_type=jnp.float32)
  o_ref[...] = h * dinv


def _mm_bn_body(x_ref, w_ref, deg_ref, cs_ref, css_ref, g_ref, be_ref, o_ref,
                *, inv_n):
  m = cs_ref[...] * inv_n
  v = css_ref[...] * inv_n - m * m
  s = g_ref[...] * lax.rsqrt(v + _EPSV)
  t = be_ref[...] - m * s
  xb = x_ref[...] * s + t
  dinv = lax.rsqrt(deg_ref[...][:, 0:1])
  h = jnp.dot(xb, w_ref[...], preferred_element_type=jnp.float32)
  o_ref[...] = h * dinv


def _post_body(p_ref, deg_ref, b_ref, r_ref, cs_ref, css_ref):
  agg = p_ref[0] + p_ref[1]
  dinv = lax.rsqrt(deg_ref[...][:, 0:1])
  rr = jnp.maximum(agg * dinv + b_ref[...], 0.0)
  r_ref[...] = rr

  @pl.when(pl.program_id(0) == 0)
  def _():
    cs_ref[...] = jnp.zeros_like(cs_ref)
    css_ref[...] = jnp.zeros_like(css_ref)

  cs_ref[...] += jnp.sum(rr, axis=0, keepdims=True)
  css_ref[...] += jnp.sum(rr * rr, axis=0, keepdims=True)


def _post4_body(p_ref, deg_ref, b_ref, batch_ref, pooled_ref, cnt_ref, cs_ref,
                css_ref):
  agg = p_ref[0] + p_ref[1]
  dinv = lax.rsqrt(deg_ref[...][:, 0:1])
  rr = jnp.maximum(agg * dinv + b_ref[...], 0.0)
  bt = batch_ref[0]
  gi = lax.broadcasted_iota(jnp.int32, (_G, bt.shape[1]), 0)
  mt = (gi == bt).astype(jnp.float32)
  pp = jnp.dot(mt, rr, preferred_element_type=jnp.float32)
  cp = jnp.sum(mt, axis=1, keepdims=True)

  @pl.when(pl.program_id(0) == 0)
  def _():
    pooled_ref[...] = jnp.zeros_like(pooled_ref)
    cnt_ref[...] = jnp.zeros_like(cnt_ref)
    cs_ref[...] = jnp.zeros_like(cs_ref)
    css_ref[...] = jnp.zeros_like(css_ref)

  pooled_ref[...] += pp
  cnt_ref[...] += cp
  cs_ref[...] += jnp.sum(rr, axis=0, keepdims=True)
  css_ref[...] += jnp.sum(rr * rr, axis=0, keepdims=True)


def _head_body(pooled_ref, cnt_ref, cs_ref, css_ref, g4_ref, be4_ref,
               fw1_ref, fb1_ref, fg1_ref, fbe1_ref, fw2_ref, fb2_ref,
               fw3_ref, fb3_ref, o_ref, *, inv_n, g):
  m = cs_ref[...] * inv_n
  v = css_ref[...] * inv_n - m * m
  s = g4_ref[...] * lax.rsqrt(v + _EPSV)
  t = be4_ref[...] - m * s
  pooled = pooled_ref[...] * s + cnt_ref[...] * t
  z = jnp.dot(pooled, fw1_ref[...], preferred_element_type=jnp.float32)
  z = jnp.maximum(z + fb1_ref[...], 0.0)
  m1 = jnp.sum(z, axis=0, keepdims=True) * (1.0 / g)
  v1 = jnp.sum(z * z, axis=0, keepdims=True) * (1.0 / g) - m1 * m1
  z = (z - m1) * lax.rsqrt(v1 + _EPSV) * fg1_ref[...] + fbe1_ref[...]
  z = jnp.dot(z, fw2_ref[...], preferred_element_type=jnp.float32)
  z = jnp.maximum(z + fb2_ref[...], 0.0)
  o_ref[...] = (jnp.dot(z, fw3_ref[...], preferred_element_type=jnp.float32)
                + fb3_ref[...])


def _row_spec(rb, d):
  return pl.BlockSpec((rb, d), lambda i: (i, 0))


def _full_spec(shape):
  nd = len(shape)
  return pl.BlockSpec(shape, lambda *i: (0,) * nd)


def _mm(x, w, deg):
  n, d = x.shape
  return pl.pallas_call(
      _mm_body,
      grid=(n // _RB,),
      in_specs=[_row_spec(_RB, d), _full_spec(w.shape), _row_spec(_RB, 8)],
      out_specs=_row_spec(_RB, w.shape[1]),
      out_shape=jax.ShapeDtypeStruct((n, w.shape[1]), jnp.float32),
  )(x, w, deg)


def _mm_bn(x, w, deg, cs, css, gr, ber):
  n, d = x.shape
  return pl.pallas_call(
      functools.partial(_mm_bn_body, inv_n=1.0 / n),
      grid=(n // _RB,),
      in_specs=[_row_spec(_RB, d), _full_spec(w.shape), _row_spec(_RB, 8),
                _full_spec(cs.shape), _full_spec(css.shape),
                _full_spec(gr.shape), _full_spec(ber.shape)],
      out_specs=_row_spec(_RB, w.shape[1]),
      out_shape=jax.ShapeDtypeStruct((n, w.shape[1]), jnp.float32),
  )(x, w, deg, cs, css, gr, ber)


def _post(p, deg, br):
  _, n, d = p.shape
  return pl.pallas_call(
      _post_body,
      grid=(n // _RB,),
      in_specs=[pl.BlockSpec((2, _RB, d), lambda i: (0, i, 0)),
                _row_spec(_RB, 8), _full_spec(br.shape)],
      out_specs=[_row_spec(_RB, d), _full_spec((1, d)), _full_spec((1, d))],
      out_shape=[jax.ShapeDtypeStruct((n, d), jnp.float32),
                 jax.ShapeDtypeStruct((1, d), jnp.float32),
                 jax.ShapeDtypeStruct((1, d), jnp.float32)],
  )(p, deg, br)


def _post4(p, deg, br, batch3):
  _, n, d = p.shape
  return pl.pallas_call(
      _post4_body,
      grid=(n // _RB,),
      in_specs=[pl.BlockSpec((2, _RB, d), lambda i: (0, i, 0)),
                _row_spec(_RB, 8), _full_spec(br.shape),
                pl.BlockSpec((1, 1, _RB), lambda i: (i, 0, 0))],
      out_specs=[_full_spec((_G, d)), _full_spec((_G, 1)),
                 _full_spec((1, d)), _full_spec((1, d))],
      out_shape=[jax.ShapeDtypeStruct((_G, d), jnp.float32),
                 jax.ShapeDtypeStruct((_G, 1), jnp.float32),
                 jax.ShapeDtypeStruct((1, d), jnp.float32),
                 jax.ShapeDtypeStruct((1, d), jnp.float32)],
  )(p, deg, br, batch3)


def _head(pooled, cnt, cs, css, g4r, be4r, fw1, fb1r, fg1r, fbe1r, fw2, fb2r,
          fw3, fb3r, n_nodes):
  args = [pooled, cnt, cs, css, g4r, be4r, fw1, fb1r, fg1r, fbe1r, fw2, fb2r,
          fw3, fb3r]
  return pl.pallas_call(
      functools.partial(_head_body, inv_n=1.0 / n_nodes, g=float(_G)),
      in_specs=[_full_spec(a.shape) for a in args],
      out_specs=_full_spec((_G, 1)),
      out_shape=jax.ShapeDtypeStruct((_G, 1), jnp.float32),
  )(*args)


def kernel(x, edge_index, batch, W1, b1, g1, be1, W2, b2, g2, be2, W3, b3,
           g3, be3, W4, b4, g4, be4, fW1, fb1, fg1, fbe1, fW2, fb2, fW3, fb3):
  n, d = x.shape
  e = edge_index.shape[1]
  nw = _NC * _NS

  src = edge_index[0].reshape(nw, e // (nw * _C), _C)
  dst = edge_index[1].reshape(nw, e // (nw * _C), _C)
  zeros_nd = jnp.zeros((n, d), jnp.float32)
  ones8 = jnp.ones((n, 8), jnp.float32)
  zeros8 = jnp.zeros((n, 8), jnp.float32)
  batch3 = batch.reshape(n // _RB, 1, _RB)
  row = lambda a: a.reshape(1, -1)

  scat = _make_scatter(n, e, d)
  deg = _deg8sum(_make_deg(n, e)(dst, ones8, zeros8))

  hs = _mm(x, W1, deg)
  p = scat(src, dst, hs, zeros_nd)
  r, cs, css = _post(p, deg, row(b1))

  hs = _mm_bn(r, W2, deg, cs, css, row(g1), row(be1))
  p = scat(src, dst, hs, zeros_nd)
  r, cs, css = _post(p, deg, row(b2))

  hs = _mm_bn(r, W3, deg, cs, css, row(g2), row(be2))
  p = scat(src, dst, hs, zeros_nd)
  r, cs, css = _post(p, deg, row(b3))

  hs = _mm_bn(r, W4, deg, cs, css, row(g3), row(be3))
  p = scat(src, dst, hs, zeros_nd)
  pooled, cnt, cs, css = _post4(p, deg, row(b4), batch3)

  return _head(pooled, cnt, cs, css, row(g4), row(be4), fW1, row(fb1),
               row(fg1), row(fbe1), fW2, row(fb2), fW3, row(fb3), n)
